# baseline (speedup 1.0000x reference)
import jax
import jax.numpy as jnp
from jax import lax
from jax.experimental import pallas as pl
from jax.experimental.pallas import tpu as pltpu
from jax.experimental.pallas import tpu_sc as plsc

N = 10000
D = 128
E = 320000

_NC = 2
_NS = 16
_NW = _NC * _NS

_CH = 64
_CPT = 160
_NCHUNK = 5056
_NCHUNK_PAD = _CPT * _NW
_NACC = 10112
_NPT = _NACC // _NS
_TRASH = 10016

_mesh = plsc.VectorSubcoreMesh(core_axis_name="c", subcore_axis_name="s",
                               num_cores=_NC, num_subcores=_NS)


def _idx_ops(src_hbm, dst_hbm, base, sidx, didx, semi):
    def start(jj, q):
        off = (base + jj) * _CH
        pltpu.async_copy(src_hbm.at[pl.ds(off, _CH)], sidx[q], semi[q])
        pltpu.async_copy(dst_hbm.at[pl.ds(off, _CH)], didx[q], semi[q])

    def wait(q):
        pltpu.make_async_copy(src_hbm.at[pl.ds(0, _CH)], sidx[q], semi[q]).wait()
        pltpu.make_async_copy(dst_hbm.at[pl.ds(0, _CH)], didx[q], semi[q]).wait()

    return start, wait


def _agg_body(src_hbm, dst_hbm, h_hbm, zeros_hbm, out_hbm, *refs):
    sidx, didx = list(refs[0:8]), list(refs[8:16])
    rows = list(refs[16:20])
    acc = refs[20]
    semi = list(refs[21:29])
    semg = list(refs[29:33])
    sems = list(refs[33:37])
    c = lax.axis_index("c")
    s = lax.axis_index("s")
    wid = c * _NS + s
    nb = jnp.minimum(_CPT, jnp.maximum(0, _NCHUNK - wid * _CPT))
    base = wid * _CPT
    idx_start, idx_wait = _idx_ops(src_hbm, dst_hbm, base, sidx, didx, semi)

    def gather_start(q, r):
        pltpu.async_copy(h_hbm.at[sidx[q]], rows[r], semg[r])

    def gather_wait(q, r):
        pltpu.make_async_copy(h_hbm.at[sidx[q]], rows[r], semg[r]).wait()

    def scat_start(q, r):
        pltpu.async_copy(rows[r], acc.at[didx[q]], sems[r], add=True)

    def scat_wait(q, r):
        pltpu.make_async_copy(rows[r], acc.at[didx[q]], sems[r]).wait()

    for q in range(4):
        idx_start(q, q)
    pltpu.sync_copy(zeros_hbm.at[pl.ds(s * _NPT, _NPT)],
                    acc.at[pl.ds(s * _NPT, _NPT)])
    plsc.subcore_barrier()

    def group(g, carry):
        for b in range(8):
            j = 8 * g + b
            q, r = b, b % 4
            q1, r1 = (b - 1) % 8, (b - 1) % 4
            idx_wait(q)
            if b >= 4:
                scat_wait(q - 4, r)
            else:
                @pl.when(g > 0)
                def _():
                    scat_wait(q + 4, r)
            gather_start(q, r)

            @pl.when(j + 4 < nb)
            def _():
                idx_start(j + 4, (b + 4) % 8)

            if b >= 1:
                gather_wait(q1, r1)
                scat_start(q1, r1)
            else:
                @pl.when(g > 0)
                def _():
                    gather_wait(q1, r1)
                    scat_start(q1, r1)
        return carry

    lax.fori_loop(0, nb // 8, group, 0)

    gather_wait(7, 3)
    scat_start(7, 3)
    scat_wait(4, 0)
    scat_wait(5, 1)
    scat_wait(6, 2)
    scat_wait(7, 3)

    plsc.subcore_barrier()
    pltpu.sync_copy(acc.at[pl.ds(s * _NPT, _NPT)],
                    out_hbm.at[c].at[pl.ds(s * _NPT, _NPT)])


_agg_call = pl.kernel(
    _agg_body,
    out_type=jax.ShapeDtypeStruct((_NC, _NACC, D), jnp.float32),
    mesh=_mesh,
    scratch_types=(
        [pltpu.VMEM((_CH,), jnp.int32)] * 16
        + [pltpu.VMEM((_CH, D), jnp.float32)] * 4
        + [pltpu.VMEM_SHARED((_NACC, D), jnp.float32)]
        + [pltpu.SemaphoreType.DMA] * 16
    ),
)


def _deg_body(dst_hbm, ones_hbm, zeros_hbm, out_hbm, *refs):
    didx = list(refs[0:8])
    ones_v = refs[8]
    acc = refs[9]
    semi = list(refs[10:18])
    sems = list(refs[18:22])
    c = lax.axis_index("c")
    s = lax.axis_index("s")
    wid = c * _NS + s
    nb = jnp.minimum(_CPT, jnp.maximum(0, _NCHUNK - wid * _CPT))
    base = wid * _CPT

    def idx_start(jj, q):
        off = (base + jj) * _CH
        pltpu.async_copy(dst_hbm.at[pl.ds(off, _CH)], didx[q], semi[q])

    def idx_wait(q):
        pltpu.make_async_copy(dst_hbm.at[pl.ds(0, _CH)], didx[q], semi[q]).wait()

    def scat_start(q, r):
        pltpu.async_copy(ones_v, acc.at[didx[q]], sems[r], add=True)

    def scat_wait(q, r):
        pltpu.make_async_copy(ones_v, acc.at[didx[q]], sems[r]).wait()

    for q in range(4):
        idx_start(q, q)
    pltpu.sync_copy(ones_hbm, ones_v)
    pltpu.sync_copy(zeros_hbm.at[pl.ds(s * _NPT, _NPT)],
                    acc.at[pl.ds(s * _NPT, _NPT)])
    plsc.subcore_barrier()

    def group(g, carry):
        for b in range(8):
            j = 8 * g + b
            q, r = b, b % 4
            idx_wait(q)
            if b >= 4:
                scat_wait(q - 4, r)
            else:
                @pl.when(g > 0)
                def _():
                    scat_wait(q + 4, r)
            scat_start(q, r)

            @pl.when(j + 4 < nb)
            def _():
                idx_start(j + 4, (b + 4) % 8)
        return carry

    lax.fori_loop(0, nb // 8, group, 0)
    scat_wait(4, 0)
    scat_wait(5, 1)
    scat_wait(6, 2)
    scat_wait(7, 3)

    plsc.subcore_barrier()
    pltpu.sync_copy(acc.at[pl.ds(s * _NPT, _NPT)],
                    out_hbm.at[c].at[pl.ds(s * _NPT, _NPT)])


_deg_call = pl.kernel(
    _deg_body,
    out_type=jax.ShapeDtypeStruct((_NC, _NACC, D), jnp.float32),
    mesh=_mesh,
    scratch_types=(
        [pltpu.VMEM((_CH,), jnp.int32)] * 8
        + [pltpu.VMEM((_CH, D), jnp.float32)]
        + [pltpu.VMEM_SHARED((_NACC, D), jnp.float32)]
        + [pltpu.SemaphoreType.DMA] * 12
    ),
)


_BLK = 2000


def _dinv_of(dega_ref, degb_ref):
    deg = 1.0 + dega_ref[:, 0:1] + degb_ref[:, 0:1]
    return jnp.where(deg > 0, lax.rsqrt(jnp.maximum(deg, 1e-12)), 0.0)


def _b1_body(x_ref, w_ref, dega_ref, degb_ref, h_ref):
    dinv = _dinv_of(dega_ref, degb_ref)
    h = jnp.dot(x_ref[...], w_ref[...], preferred_element_type=jnp.float32)
    h_ref[...] = h * dinv


def _b2_body(agg_ref, h1s_ref, dega_ref, degb_ref, b1_ref, gamma_ref,
             beta_ref, mean_ref, var_ref, w2_ref, h2s_ref):
    dinv = _dinv_of(dega_ref, degb_ref)
    z = dinv * (agg_ref[0] + agg_ref[1] + h1s_ref[...]) + b1_ref[...]
    y = gamma_ref[...] * (z - mean_ref[...]) * lax.rsqrt(var_ref[...] + 1e-5)
    y = jnp.maximum(y + beta_ref[...], 0.0)
    h2s_ref[...] = jnp.dot(y, w2_ref[...],
                           preferred_element_type=jnp.float32) * dinv


def _b3_body(agg_ref, h2s_ref, dega_ref, degb_ref, b2_ref, out_ref):
    dinv = _dinv_of(dega_ref, degb_ref)
    out_ref[...] = dinv * (agg_ref[0] + agg_ref[1] + h2s_ref[...]) + b2_ref[...]


_row_spec = pl.BlockSpec((_BLK, D), lambda i: (i, 0))
_deg_spec = pl.BlockSpec((_BLK, D), lambda i: (i, 0))
_mat_spec = pl.BlockSpec((D, D), lambda i: (0, 0))
_vec_spec = pl.BlockSpec((1, D), lambda i: (0, 0))
_agg_spec = pl.BlockSpec((_NC, _BLK, D), lambda i: (0, i, 0))

_b1_call = pl.pallas_call(
    _b1_body,
    grid=(N // _BLK,),
    in_specs=[_row_spec, _mat_spec, _deg_spec, _deg_spec],
    out_specs=_row_spec,
    out_shape=jax.ShapeDtypeStruct((N, D), jnp.float32),
)

_b2_call = pl.pallas_call(
    _b2_body,
    grid=(N // _BLK,),
    in_specs=[_agg_spec, _row_spec, _deg_spec, _deg_spec, _vec_spec,
              _vec_spec, _vec_spec, _vec_spec, _vec_spec, _mat_spec],
    out_specs=_row_spec,
    out_shape=jax.ShapeDtypeStruct((N, D), jnp.float32),
)

_b3_call = pl.pallas_call(
    _b3_body,
    grid=(N // _BLK,),
    in_specs=[_agg_spec, _row_spec, _deg_spec, _deg_spec, _vec_spec],
    out_specs=_row_spec,
    out_shape=jax.ShapeDtypeStruct((N, D), jnp.float32),
)


def _pad_chunks(idx, fill):
    pad = _NCHUNK * _CH - E
    return jnp.concatenate(
        [idx.astype(jnp.int32), jnp.full((pad,), fill, jnp.int32)])


def kernel(x, edge_index, W1, b1, gamma, beta, running_mean, running_var,
           W2, b2):
    src = _pad_chunks(edge_index[0], 0)
    dst = _pad_chunks(edge_index[1], _TRASH)
    zeros_d = jnp.zeros((_NACC, D), jnp.float32)
    ones_r = jnp.ones((_CH, D), jnp.float32)

    degp = _deg_call(dst, ones_r, zeros_d)
    dega, degb = degp[0], degp[1]

    h1s = _b1_call(x, W1, dega, degb)
    agg1 = _agg_call(src, dst, h1s, zeros_d)
    h2s = _b2_call(agg1, h1s, dega, degb,
                   b1.reshape(1, D), gamma.reshape(1, D), beta.reshape(1, D),
                   running_mean.reshape(1, D), running_var.reshape(1, D), W2)
    agg2 = _agg_call(src, dst, h2s, zeros_d)
    return _b3_call(agg2, h2s, dega, degb, b2.reshape(1, D))


# speedup vs baseline: 1.0123x; 1.0123x over previous
import jax
import jax.numpy as jnp
from jax import lax
from jax.experimental import pallas as pl
from jax.experimental.pallas import tpu as pltpu
from jax.experimental.pallas import tpu_sc as plsc

N = 10000
D = 128
E = 320000

_NC = 2
_NS = 16
_NW = _NC * _NS

_CH = 128
_CPT = 80
_NCHUNK = 2528
_NCHUNK_PAD = _CPT * _NW
_NACC = 10112
_NPT = _NACC // _NS
_TRASH = 10016

_mesh = plsc.VectorSubcoreMesh(core_axis_name="c", subcore_axis_name="s",
                               num_cores=_NC, num_subcores=_NS)


def _idx_ops(src_hbm, dst_hbm, base, sidx, didx, semi):
    def start(jj, q):
        off = (base + jj) * _CH
        pltpu.async_copy(src_hbm.at[pl.ds(off, _CH)], sidx[q], semi[q])
        pltpu.async_copy(dst_hbm.at[pl.ds(off, _CH)], didx[q], semi[q])

    def wait(q):
        pltpu.make_async_copy(src_hbm.at[pl.ds(0, _CH)], sidx[q], semi[q]).wait()
        pltpu.make_async_copy(dst_hbm.at[pl.ds(0, _CH)], didx[q], semi[q]).wait()

    return start, wait


def _agg_body(src_hbm, dst_hbm, h_hbm, zeros_hbm, out_hbm, *refs):
    sidx, didx = list(refs[0:4]), list(refs[4:8])
    rows = list(refs[8:10])
    acc = refs[10]
    semi = list(refs[11:15])
    semg = list(refs[15:17])
    sems = list(refs[17:19])
    c = lax.axis_index("c")
    s = lax.axis_index("s")
    wid = c * _NS + s
    nb = jnp.minimum(_CPT, jnp.maximum(0, _NCHUNK - wid * _CPT))
    base = wid * _CPT
    idx_start, idx_wait = _idx_ops(src_hbm, dst_hbm, base, sidx, didx, semi)

    def gather_start(q, r):
        pltpu.async_copy(h_hbm.at[sidx[q]], rows[r], semg[r])

    def gather_wait(q, r):
        pltpu.make_async_copy(h_hbm.at[sidx[q]], rows[r], semg[r]).wait()

    def scat_start(q, r):
        pltpu.async_copy(rows[r], acc.at[didx[q]], sems[r], add=True)

    def scat_wait(q, r):
        pltpu.make_async_copy(rows[r], acc.at[didx[q]], sems[r]).wait()

    idx_start(0, 0)
    idx_start(1, 1)
    pltpu.sync_copy(zeros_hbm.at[pl.ds(s * _NPT, _NPT)],
                    acc.at[pl.ds(s * _NPT, _NPT)])
    plsc.subcore_barrier()

    def group(g, carry):
        for b in range(4):
            j = 4 * g + b
            q, r = b, b % 2
            q1, r1 = (b - 1) % 4, (b - 1) % 2
            idx_wait(q)
            if b >= 2:
                scat_wait(q, r)
            else:
                @pl.when(g > 0)
                def _():
                    scat_wait(q, r)
            gather_start(q, r)

            @pl.when(j + 2 < nb)
            def _():
                idx_start(j + 2, (b + 2) % 4)

            if b >= 1:
                gather_wait(q1, r1)
                scat_start(q1, r1)
            else:
                @pl.when(g > 0)
                def _():
                    gather_wait(q1, r1)
                    scat_start(q1, r1)
        return carry

    lax.fori_loop(0, nb // 4, group, 0)

    gather_wait(3, 1)
    scat_start(3, 1)
    scat_wait(2, 0)
    scat_wait(3, 1)

    plsc.subcore_barrier()
    pltpu.sync_copy(acc.at[pl.ds(s * _NPT, _NPT)],
                    out_hbm.at[c].at[pl.ds(s * _NPT, _NPT)])


_agg_call = pl.kernel(
    _agg_body,
    out_type=jax.ShapeDtypeStruct((_NC, _NACC, D), jnp.float32),
    mesh=_mesh,
    scratch_types=(
        [pltpu.VMEM((_CH,), jnp.int32)] * 8
        + [pltpu.VMEM((_CH, D), jnp.float32)] * 2
        + [pltpu.VMEM_SHARED((_NACC, D), jnp.float32)]
        + [pltpu.SemaphoreType.DMA] * 8
    ),
)


def _deg_body(dst_hbm, ones_hbm, zeros_hbm, out_hbm, *refs):
    didx = list(refs[0:4])
    ones_v = refs[4]
    acc = refs[5]
    semi = list(refs[6:10])
    sems = list(refs[10:12])
    c = lax.axis_index("c")
    s = lax.axis_index("s")
    wid = c * _NS + s
    nb = jnp.minimum(_CPT, jnp.maximum(0, _NCHUNK - wid * _CPT))
    base = wid * _CPT

    def idx_start(jj, q):
        off = (base + jj) * _CH
        pltpu.async_copy(dst_hbm.at[pl.ds(off, _CH)], didx[q], semi[q])

    def idx_wait(q):
        pltpu.make_async_copy(dst_hbm.at[pl.ds(0, _CH)], didx[q], semi[q]).wait()

    def scat_start(q, r):
        pltpu.async_copy(ones_v, acc.at[didx[q]], sems[r], add=True)

    def scat_wait(q, r):
        pltpu.make_async_copy(ones_v, acc.at[didx[q]], sems[r]).wait()

    idx_start(0, 0)
    idx_start(1, 1)
    pltpu.sync_copy(ones_hbm, ones_v)
    pltpu.sync_copy(zeros_hbm.at[pl.ds(s * _NPT, _NPT)],
                    acc.at[pl.ds(s * _NPT, _NPT)])
    plsc.subcore_barrier()

    def group(g, carry):
        for b in range(4):
            j = 4 * g + b
            q, r = b, b % 2
            idx_wait(q)
            if b >= 2:
                scat_wait(q, r)
            else:
                @pl.when(g > 0)
                def _():
                    scat_wait(q, r)
            scat_start(q, r)

            @pl.when(j + 2 < nb)
            def _():
                idx_start(j + 2, (b + 2) % 4)
        return carry

    lax.fori_loop(0, nb // 4, group, 0)
    scat_wait(2, 0)
    scat_wait(3, 1)

    plsc.subcore_barrier()
    pltpu.sync_copy(acc.at[pl.ds(s * _NPT, _NPT)],
                    out_hbm.at[c].at[pl.ds(s * _NPT, _NPT)])


_deg_call = pl.kernel(
    _deg_body,
    out_type=jax.ShapeDtypeStruct((_NC, _NACC, D), jnp.float32),
    mesh=_mesh,
    scratch_types=(
        [pltpu.VMEM((_CH,), jnp.int32)] * 4
        + [pltpu.VMEM((_CH, D), jnp.float32)]
        + [pltpu.VMEM_SHARED((_NACC, D), jnp.float32)]
        + [pltpu.SemaphoreType.DMA] * 6
    ),
)


_BLK = 2000


def _dinv_of(dega_ref, degb_ref):
    deg = 1.0 + dega_ref[:, 0:1] + degb_ref[:, 0:1]
    return jnp.where(deg > 0, lax.rsqrt(jnp.maximum(deg, 1e-12)), 0.0)


def _b1_body(x_ref, w_ref, dega_ref, degb_ref, h_ref):
    dinv = _dinv_of(dega_ref, degb_ref)
    h = jnp.dot(x_ref[...], w_ref[...], preferred_element_type=jnp.float32)
    h_ref[...] = h * dinv


def _b2_body(agg_ref, h1s_ref, dega_ref, degb_ref, b1_ref, gamma_ref,
             beta_ref, mean_ref, var_ref, w2_ref, h2s_ref):
    dinv = _dinv_of(dega_ref, degb_ref)
    z = dinv * (agg_ref[0] + agg_ref[1] + h1s_ref[...]) + b1_ref[...]
    y = gamma_ref[...] * (z - mean_ref[...]) * lax.rsqrt(var_ref[...] + 1e-5)
    y = jnp.maximum(y + beta_ref[...], 0.0)
    h2s_ref[...] = jnp.dot(y, w2_ref[...],
                           preferred_element_type=jnp.float32) * dinv


def _b3_body(agg_ref, h2s_ref, dega_ref, degb_ref, b2_ref, out_ref):
    dinv = _dinv_of(dega_ref, degb_ref)
    out_ref[...] = dinv * (agg_ref[0] + agg_ref[1] + h2s_ref[...]) + b2_ref[...]


_row_spec = pl.BlockSpec((_BLK, D), lambda i: (i, 0))
_deg_spec = pl.BlockSpec((_BLK, D), lambda i: (i, 0))
_mat_spec = pl.BlockSpec((D, D), lambda i: (0, 0))
_vec_spec = pl.BlockSpec((1, D), lambda i: (0, 0))
_agg_spec = pl.BlockSpec((_NC, _BLK, D), lambda i: (0, i, 0))

_b1_call = pl.pallas_call(
    _b1_body,
    grid=(N // _BLK,),
    in_specs=[_row_spec, _mat_spec, _deg_spec, _deg_spec],
    out_specs=_row_spec,
    out_shape=jax.ShapeDtypeStruct((N, D), jnp.float32),
)

_b2_call = pl.pallas_call(
    _b2_body,
    grid=(N // _BLK,),
    in_specs=[_agg_spec, _row_spec, _deg_spec, _deg_spec, _vec_spec,
              _vec_spec, _vec_spec, _vec_spec, _vec_spec, _mat_spec],
    out_specs=_row_spec,
    out_shape=jax.ShapeDtypeStruct((N, D), jnp.float32),
)

_b3_call = pl.pallas_call(
    _b3_body,
    grid=(N // _BLK,),
    in_specs=[_agg_spec, _row_spec, _deg_spec, _deg_spec, _vec_spec],
    out_specs=_row_spec,
    out_shape=jax.ShapeDtypeStruct((N, D), jnp.float32),
)


def _pad_chunks(idx, fill):
    pad = _NCHUNK * _CH - E
    return jnp.concatenate(
        [idx.astype(jnp.int32), jnp.full((pad,), fill, jnp.int32)])


def kernel(x, edge_index, W1, b1, gamma, beta, running_mean, running_var,
           W2, b2):
    src = _pad_chunks(edge_index[0], 0)
    dst = _pad_chunks(edge_index[1], _TRASH)
    zeros_d = jnp.zeros((_NACC, D), jnp.float32)
    ones_r = jnp.ones((_CH, D), jnp.float32)

    degp = _deg_call(dst, ones_r, zeros_d)
    dega, degb = degp[0], degp[1]

    h1s = _b1_call(x, W1, dega, degb)
    agg1 = _agg_call(src, dst, h1s, zeros_d)
    h2s = _b2_call(agg1, h1s, dega, degb,
                   b1.reshape(1, D), gamma.reshape(1, D), beta.reshape(1, D),
                   running_mean.reshape(1, D), running_var.reshape(1, D), W2)
    agg2 = _agg_call(src, dst, h2s, zeros_d)
    return _b3_call(agg2, h2s, dega, degb, b2.reshape(1, D))


# speedup vs baseline: 1.9573x; 1.9334x over previous
import jax
import jax.numpy as jnp
from jax import lax
from jax.experimental import pallas as pl
from jax.experimental.pallas import tpu as pltpu
from jax.experimental.pallas import tpu_sc as plsc

N = 10000
D = 128
E = 320000

_NC = 2
_NS = 16
_NW = _NC * _NS

_CH = 128
_CPT = 80
_NCHUNK = 2500
_NCHUNK_PAD = _CPT * _NW
_NACC = 10112
_NPT = _NACC // _NS
_TRASH = 10016

_mesh = plsc.VectorSubcoreMesh(core_axis_name="c", subcore_axis_name="s",
                               num_cores=_NC, num_subcores=_NS)


def _idx_ops(src_hbm, dst_hbm, base, sidx, didx, semi):
    def start(jj, q):
        off = (base + jj) * _CH
        pltpu.async_copy(src_hbm.at[pl.ds(off, _CH)], sidx[q], semi[q])
        pltpu.async_copy(dst_hbm.at[pl.ds(off, _CH)], didx[q], semi[q])

    def wait(q):
        pltpu.make_async_copy(src_hbm.at[pl.ds(0, _CH)], sidx[q], semi[q]).wait()
        pltpu.make_async_copy(dst_hbm.at[pl.ds(0, _CH)], didx[q], semi[q]).wait()

    return start, wait


def _agg_body(src_hbm, dst_hbm, h_hbm, zeros_hbm, out_hbm, *refs):
    sidx, didx = list(refs[0:4]), list(refs[4:8])
    rows = list(refs[8:10])
    acc = refs[10]
    semi = list(refs[11:15])
    semg = list(refs[15:17])
    sems = list(refs[17:19])
    c = lax.axis_index("c")
    s = lax.axis_index("s")
    wid = c * _NS + s
    nb = jnp.minimum(_CPT, jnp.maximum(0, _NCHUNK - wid * _CPT))
    base = wid * _CPT
    idx_start, idx_wait = _idx_ops(src_hbm, dst_hbm, base, sidx, didx, semi)

    def gather_start(q, r):
        pltpu.async_copy(h_hbm.at[sidx[q]], rows[r], semg[r])

    def gather_wait(q, r):
        pltpu.make_async_copy(h_hbm.at[sidx[q]], rows[r], semg[r]).wait()

    def scat_start(q, r):
        pltpu.async_copy(rows[r], acc.at[didx[q]], sems[r], add=True)

    def scat_wait(q, r):
        pltpu.make_async_copy(rows[r], acc.at[didx[q]], sems[r]).wait()

    idx_start(0, 0)
    idx_start(1, 1)
    pltpu.sync_copy(zeros_hbm.at[pl.ds(s * _NPT, _NPT)],
                    acc.at[pl.ds(s * _NPT, _NPT)])
    plsc.subcore_barrier()

    def group(g, carry):
        for b in range(4):
            j = 4 * g + b
            q, r = b, b % 2
            q1, r1 = (b - 1) % 4, (b - 1) % 2
            idx_wait(q)
            if b >= 2:
                scat_wait(q, r)
            else:
                @pl.when(g > 0)
                def _():
                    scat_wait(q, r)
            gather_start(q, r)

            @pl.when(j + 2 < nb)
            def _():
                idx_start(j + 2, (b + 2) % 4)

            if b >= 1:
                gather_wait(q1, r1)
                scat_start(q1, r1)
            else:
                @pl.when(g > 0)
                def _():
                    gather_wait(q1, r1)
                    scat_start(q1, r1)
        return carry

    lax.fori_loop(0, nb // 4, group, 0)

    gather_wait(3, 1)
    scat_start(3, 1)
    scat_wait(2, 0)
    scat_wait(3, 1)

    plsc.subcore_barrier()
    pltpu.sync_copy(acc.at[pl.ds(s * _NPT, _NPT)],
                    out_hbm.at[c].at[pl.ds(s * _NPT, _NPT)])


_agg_call = pl.kernel(
    _agg_body,
    out_type=jax.ShapeDtypeStruct((_NC, _NACC, D), jnp.float32),
    mesh=_mesh,
    scratch_types=(
        [pltpu.VMEM((_CH,), jnp.int32)] * 8
        + [pltpu.VMEM((_CH, D), jnp.float32)] * 2
        + [pltpu.VMEM_SHARED((_NACC, D), jnp.float32)]
        + [pltpu.SemaphoreType.DMA] * 8
    ),
)


def _deg_body(dst_hbm, ones_hbm, zeros_hbm, out_hbm, *refs):
    didx = list(refs[0:4])
    ones_v = refs[4]
    acc = refs[5]
    semi = list(refs[6:10])
    sems = list(refs[10:12])
    c = lax.axis_index("c")
    s = lax.axis_index("s")
    wid = c * _NS + s
    nb = jnp.minimum(_CPT, jnp.maximum(0, _NCHUNK - wid * _CPT))
    base = wid * _CPT

    def idx_start(jj, q):
        off = (base + jj) * _CH
        pltpu.async_copy(dst_hbm.at[pl.ds(off, _CH)], didx[q], semi[q])

    def idx_wait(q):
        pltpu.make_async_copy(dst_hbm.at[pl.ds(0, _CH)], didx[q], semi[q]).wait()

    def scat_start(q, r):
        pltpu.async_copy(ones_v, acc.at[didx[q]], sems[r], add=True)

    def scat_wait(q, r):
        pltpu.make_async_copy(ones_v, acc.at[didx[q]], sems[r]).wait()

    idx_start(0, 0)
    idx_start(1, 1)
    pltpu.sync_copy(ones_hbm, ones_v)
    pltpu.sync_copy(zeros_hbm.at[pl.ds(s * _NPT, _NPT)],
                    acc.at[pl.ds(s * _NPT, _NPT)])
    plsc.subcore_barrier()

    def group(g, carry):
        for b in range(4):
            j = 4 * g + b
            q, r = b, b % 2
            idx_wait(q)
            if b >= 2:
                scat_wait(q, r)
            else:
                @pl.when(g > 0)
                def _():
                    scat_wait(q, r)
            scat_start(q, r)

            @pl.when(j + 2 < nb)
            def _():
                idx_start(j + 2, (b + 2) % 4)
        return carry

    lax.fori_loop(0, nb // 4, group, 0)
    scat_wait(2, 0)
    scat_wait(3, 1)

    plsc.subcore_barrier()
    pltpu.sync_copy(acc.at[pl.ds(s * _NPT, _NPT)],
                    out_hbm.at[c].at[pl.ds(s * _NPT, _NPT)])


_deg_call = pl.kernel(
    _deg_body,
    out_type=jax.ShapeDtypeStruct((_NC, _NACC, D), jnp.float32),
    mesh=_mesh,
    scratch_types=(
        [pltpu.VMEM((_CH,), jnp.int32)] * 4
        + [pltpu.VMEM((_CH, D), jnp.float32)]
        + [pltpu.VMEM_SHARED((_NACC, D), jnp.float32)]
        + [pltpu.SemaphoreType.DMA] * 6
    ),
)


_BLK = 2000


def _dinv_of(dega_ref, degb_ref):
    deg = 1.0 + dega_ref[:, 0:1] + degb_ref[:, 0:1]
    return jnp.where(deg > 0, lax.rsqrt(jnp.maximum(deg, 1e-12)), 0.0)


def _b1_body(x_ref, w_ref, dega_ref, degb_ref, h_ref):
    dinv = _dinv_of(dega_ref, degb_ref)
    h = jnp.dot(x_ref[...], w_ref[...], preferred_element_type=jnp.float32)
    h_ref[...] = h * dinv


def _b2_body(agg_ref, h1s_ref, dega_ref, degb_ref, b1_ref, gamma_ref,
             beta_ref, mean_ref, var_ref, w2_ref, h2s_ref):
    dinv = _dinv_of(dega_ref, degb_ref)
    z = dinv * (agg_ref[0] + agg_ref[1] + h1s_ref[...]) + b1_ref[...]
    y = gamma_ref[...] * (z - mean_ref[...]) * lax.rsqrt(var_ref[...] + 1e-5)
    y = jnp.maximum(y + beta_ref[...], 0.0)
    h2s_ref[...] = jnp.dot(y, w2_ref[...],
                           preferred_element_type=jnp.float32) * dinv


def _b3_body(agg_ref, h2s_ref, dega_ref, degb_ref, b2_ref, out_ref):
    dinv = _dinv_of(dega_ref, degb_ref)
    out_ref[...] = dinv * (agg_ref[0] + agg_ref[1] + h2s_ref[...]) + b2_ref[...]


_row_spec = pl.BlockSpec((_BLK, D), lambda i: (i, 0))
_deg_spec = pl.BlockSpec((_BLK, D), lambda i: (i, 0))
_mat_spec = pl.BlockSpec((D, D), lambda i: (0, 0))
_vec_spec = pl.BlockSpec((1, D), lambda i: (0, 0))
_agg_spec = pl.BlockSpec((_NC, _BLK, D), lambda i: (0, i, 0))

_b1_call = pl.pallas_call(
    _b1_body,
    grid=(N // _BLK,),
    in_specs=[_row_spec, _mat_spec, _deg_spec, _deg_spec],
    out_specs=_row_spec,
    out_shape=jax.ShapeDtypeStruct((N, D), jnp.float32),
)

_b2_call = pl.pallas_call(
    _b2_body,
    grid=(N // _BLK,),
    in_specs=[_agg_spec, _row_spec, _deg_spec, _deg_spec, _vec_spec,
              _vec_spec, _vec_spec, _vec_spec, _vec_spec, _mat_spec],
    out_specs=_row_spec,
    out_shape=jax.ShapeDtypeStruct((N, D), jnp.float32),
)

_b3_call = pl.pallas_call(
    _b3_body,
    grid=(N // _BLK,),
    in_specs=[_agg_spec, _row_spec, _deg_spec, _deg_spec, _vec_spec],
    out_specs=_row_spec,
    out_shape=jax.ShapeDtypeStruct((N, D), jnp.float32),
)


def _pad_chunks(idx, fill):
    pad = _NCHUNK * _CH - E
    return jnp.concatenate(
        [idx.astype(jnp.int32), jnp.full((pad,), fill, jnp.int32)])


def kernel(x, edge_index, W1, b1, gamma, beta, running_mean, running_var,
           W2, b2):
    src = _pad_chunks(edge_index[0], 0)
    dst = _pad_chunks(edge_index[1], _TRASH)
    zeros_d = jnp.zeros((_NACC, D), jnp.float32)
    ones_r = jnp.ones((_CH, D), jnp.float32)

    degp = _deg_call(dst, ones_r, zeros_d)
    dega, degb = degp[0], degp[1]

    h1s = _b1_call(x, W1, dega, degb)
    agg1 = _agg_call(src, dst, h1s, zeros_d)
    h2s = _b2_call(agg1, h1s, dega, degb,
                   b1.reshape(1, D), gamma.reshape(1, D), beta.reshape(1, D),
                   running_mean.reshape(1, D), running_var.reshape(1, D), W2)
    agg2 = _agg_call(src, dst, h2s, zeros_d)
    return _b3_call(agg2, h2s, dega, degb, b2.reshape(1, D))
